# baseline (speedup 1.0000x reference)
import functools

import jax
import jax.numpy as jnp
from jax import lax
from jax.experimental import pallas as pl
from jax.experimental.pallas import tpu as pltpu
from jax.experimental.pallas import tpu_sc as plsc

H = 2048
INTER = 1408
E = 8
R = 256
SH = 2816
S = 2048
T1 = 128
TG = 128
NT = 40
NTP = 48
NPAD = NT * TG
NW = 32
RPW = NPAD // NW
TPW = S // NW
CC = 32


def _mm(a, b):
    return jax.lax.dot_general(a, b, (((1,), (1,)), ((), ())),
                               preferred_element_type=jnp.float32)


def _mmb(a, b):
    return jax.lax.dot_general(a.astype(jnp.bfloat16), b,
                               (((1,), (1,)), ((), ())),
                               preferred_element_type=jnp.float32)


def _silu(x):
    return x * jax.nn.sigmoid(x)


def _b(w):
    return w.astype(jnp.bfloat16)


def _k1_body(x_ref, gate_w_ref, rg_w_ref, ru_w_ref,
             srg_ref, sgu_ref, sgc_ref, sru_ref, suu_ref, suc_ref,
             srd_ref, sdu_ref, sdc_ref,
             wet_ref, rgru_ref, ysh_ref):
    i = pl.program_id(0)
    x = x_ref[...]
    logits = _mm(x, gate_w_ref[...])
    m = jnp.max(logits, axis=-1, keepdims=True)
    ex = jnp.exp(logits - m)
    sc = ex / jnp.sum(ex, axis=-1, keepdims=True)
    eidx = jax.lax.broadcasted_iota(jnp.int32, sc.shape, 1)
    m1 = jnp.max(sc, axis=-1, keepdims=True)
    i1 = jnp.min(jnp.where(sc >= m1, eidx, E), axis=-1, keepdims=True)
    sc2 = jnp.where(eidx == i1, -jnp.inf, sc)
    m2 = jnp.max(sc2, axis=-1, keepdims=True)
    i2 = jnp.min(jnp.where(sc2 >= m2, eidx, E), axis=-1, keepdims=True)
    den = m1 + m2 + 1e-20
    w1 = m1 / den
    w2 = m2 / den
    we = jnp.where(eidx == i1, w1, jnp.where(eidx == i2, w2, 0.0))
    wet_ref[:, pl.ds(i * T1, T1)] = we.T
    xb = x.astype(jnp.bfloat16)
    rgru_ref[:, :R] = _mmb(xb, rg_w_ref[...])
    rgru_ref[:, R:] = _mmb(xb, ru_w_ref[...])
    sg = _silu(_mmb(_mmb(_mmb(xb, srg_ref[...]), sgu_ref[...]), sgc_ref[...]))
    su = _mmb(_mmb(_mmb(xb, sru_ref[...]), suu_ref[...]), suc_ref[...])
    si = sg * su
    ysh_ref[...] = _mmb(_mmb(_mmb(si, srd_ref[...]), sdu_ref[...]), sdc_ref[...])


def _k1(x, p):
    full = lambda shape: pl.BlockSpec(shape, lambda i: (0,) * len(shape))
    grid = S // T1
    return pl.pallas_call(
        _k1_body,
        grid=(grid,),
        in_specs=[
            pl.BlockSpec((T1, H), lambda i: (i, 0)),
            full((E, H)), full((R, H)), full((R, H)),
            full((R, H)), full((R, R)), full((SH, R)),
            full((R, H)), full((R, R)), full((SH, R)),
            full((R, SH)), full((R, R)), full((H, R)),
        ],
        out_specs=[
            pl.BlockSpec((E, S), lambda i: (0, 0)),
            pl.BlockSpec((T1, 2 * R), lambda i: (i, 0)),
            pl.BlockSpec((T1, H), lambda i: (i, 0)),
        ],
        out_shape=[
            jax.ShapeDtypeStruct((E, S), jnp.float32),
            jax.ShapeDtypeStruct((S, 2 * R), jnp.float32),
            jax.ShapeDtypeStruct((S, H), jnp.float32),
        ],
    )(x, p['gate_w'], _b(p['Rg']), _b(p['Ru']),
      _b(p['s_Rg']), _b(p['s_gU']), _b(p['s_gC']), _b(p['s_Ru']),
      _b(p['s_uU']), _b(p['s_uC']), _b(p['s_Rd']), _b(p['s_dU']),
      _b(p['s_dC']))


def _d1_body(wet_ref, gidx_ref, roww_ref, teid_ref, invlo_ref, invhi_ref,
             wet_v, tok_v, w_v, invlo_v, invhi_v, te_v):
    cid = lax.axis_index("c")
    sid = lax.axis_index("s")

    @pl.when(jnp.logical_and(cid == 0, sid <= 9))
    def _work():
        pltpu.sync_copy(wet_ref, wet_v)

        def _count(e):
            def step(j, c):
                v = wet_v[e, pl.ds(j * 16, 16)]
                return c + jnp.sum((v > 0.0).astype(jnp.int32))
            return lax.fori_loop(0, S // 16, step, jnp.int32(0))

        cnts = [_count(e) for e in range(E)]
        ntiles = [(c + (TG - 1)) // TG for c in cnts]
        toff = [jnp.int32(0)]
        for e in range(E):
            toff.append(toff[e] + ntiles[e])
        base = [t * TG for t in toff]

        for e in range(E):
            @pl.when(sid == e)
            def _compact(e=e):
                def zstep(j, _):
                    tok_v[pl.ds(j * 16, 16)] = jnp.zeros((16,), jnp.int32)
                    w_v[pl.ds(j * 16, 16)] = jnp.zeros((16,), jnp.float32)
                    return 0
                lax.fori_loop(0, (S + 16) // 16, zstep, 0)

                def step(j, cnt):
                    v = wet_v[e, pl.ds(j * 16, 16)]
                    mk = v > 0.0
                    toks = j * 16 + lax.iota(jnp.int32, 16)
                    plsc.store_compressed(tok_v.at[pl.ds(cnt, 16)], toks,
                                          mask=mk)
                    plsc.store_compressed(w_v.at[pl.ds(cnt, 16)], v, mask=mk)
                    return cnt + jnp.sum(mk.astype(jnp.int32))
                lax.fori_loop(0, S // 16, step, jnp.int32(0))

                def wstep(k, _):
                    pltpu.sync_copy(
                        tok_v.at[pl.ds(k * TG, TG)],
                        gidx_ref.at[pl.ds(base[e] + k * TG, TG)])
                    pltpu.sync_copy(
                        w_v.at[pl.ds(k * TG, TG)],
                        roww_ref.at[pl.ds(base[e] + k * TG, TG)])
                    return 0
                lax.fori_loop(0, ntiles[e], wstep, 0)

        @pl.when(sid == 8)
        def _inv():
            def step(j, carry):
                acc_s = jnp.zeros((16,), jnp.int32)
                acc_m = jnp.zeros((16,), jnp.int32)
                new = []
                for e in range(E):
                    v = wet_v[e, pl.ds(j * 16, 16)]
                    mk = v > 0.0
                    mi = mk.astype(jnp.int32)
                    pos1 = (base[e] + carry[e]) + plsc.cumsum(mi)
                    p = jnp.where(mk, pos1, 0)
                    acc_s = acc_s + p
                    acc_m = jnp.maximum(acc_m, p)
                    new.append(carry[e] + jnp.sum(mi))
                invlo_v[pl.ds(j * 16, 16)] = acc_s - acc_m
                invhi_v[pl.ds(j * 16, 16)] = acc_m
                return tuple(new)
            lax.fori_loop(0, S // 16, step, (jnp.int32(0),) * E)
            pltpu.sync_copy(invlo_v, invlo_ref)
            pltpu.sync_copy(invhi_v, invhi_ref)

        @pl.when(sid == 9)
        def _teid():
            for j in range(NTP // 16):
                iv = j * 16 + lax.iota(jnp.int32, 16)
                acc = jnp.zeros((16,), jnp.int32)
                for e in range(E):
                    acc = acc + (iv >= toff[e + 1]).astype(jnp.int32)
                te_v[pl.ds(j * 16, 16)] = jnp.minimum(acc, E - 1)
            pltpu.sync_copy(te_v, teid_ref)


def _d1(wet):
    mesh = plsc.VectorSubcoreMesh(core_axis_name="c", subcore_axis_name="s")
    f = pl.kernel(
        _d1_body,
        compiler_params=pltpu.CompilerParams(needs_layout_passes=False),
        out_type=[
            jax.ShapeDtypeStruct((NPAD,), jnp.int32),
            jax.ShapeDtypeStruct((NPAD,), jnp.float32),
            jax.ShapeDtypeStruct((NTP,), jnp.int32),
            jax.ShapeDtypeStruct((S,), jnp.int32),
            jax.ShapeDtypeStruct((S,), jnp.int32),
        ],
        mesh=mesh,
        scratch_types=[
            pltpu.VMEM((E, S), jnp.float32),
            pltpu.VMEM((S + 16,), jnp.int32),
            pltpu.VMEM((S + 16,), jnp.float32),
            pltpu.VMEM((S,), jnp.int32),
            pltpu.VMEM((S,), jnp.int32),
            pltpu.VMEM((NTP,), jnp.int32),
        ],
    )
    return f(wet)


def _d2_body(rgru_ref, gidx_ref, out_ref, idx_v, rows_v, sem0, sem1):
    cid = lax.axis_index("c")
    sid = lax.axis_index("s")
    wid = sid * 2 + cid
    rbase = wid * RPW

    pltpu.sync_copy(gidx_ref.at[pl.ds(rbase, RPW)], idx_v)
    for k in range(RPW // 16):
        idx_v[pl.ds(k * 16, 16)] = jnp.bitwise_and(
            idx_v[pl.ds(k * 16, 16)], S - 1)

    c0 = pltpu.async_copy(rgru_ref.at[idx_v.at[pl.ds(0, 128)]],
                          rows_v.at[pl.ds(0, 128)], sem0)
    c1 = pltpu.async_copy(rgru_ref.at[idx_v.at[pl.ds(128, RPW - 128)]],
                          rows_v.at[pl.ds(128, RPW - 128)], sem1)
    c0.wait()
    c1.wait()
    pltpu.sync_copy(rows_v, out_ref.at[pl.ds(rbase, RPW)])


def _d2(rgru, gidx):
    mesh = plsc.VectorSubcoreMesh(core_axis_name="c", subcore_axis_name="s")
    f = pl.kernel(
        _d2_body,
        out_type=[
            jax.ShapeDtypeStruct((NPAD, 2 * R), jnp.float32),
        ],
        mesh=mesh,
        scratch_types=[
            pltpu.VMEM((RPW,), jnp.int32),
            pltpu.VMEM((RPW, 2 * R), jnp.float32),
            pltpu.SemaphoreType.DMA,
            pltpu.SemaphoreType.DMA,
        ],
    )
    return f(rgru, gidx)[0]


def _k2_body(eid_ref, rgru_ref, roww_ref, gu_ref, gc_ref, uu_ref,
             uc_ref, rd_ref, du_ref, dc_ref, out_ref):
    w = jnp.reshape(roww_ref[0, 0, :], (TG, 1))
    gate = _silu(_mmb(_mmb(rgru_ref[:, :R], gu_ref[0]), gc_ref[0]))
    up = _mmb(_mmb(rgru_ref[:, R:] * w, uu_ref[0]), uc_ref[0])
    inter = gate * up
    y = _mmb(_mmb(_mmb(inter, rd_ref[...]), du_ref[0]), dc_ref[0])
    out_ref[...] = y.astype(jnp.bfloat16)


def _k2(rgru_s, roww3, tile_eid, p):
    grid_spec = pltpu.PrefetchScalarGridSpec(
        num_scalar_prefetch=1,
        grid=(NT,),
        in_specs=[
            pl.BlockSpec((TG, 2 * R), lambda i, eid: (i, 0)),
            pl.BlockSpec((1, 1, TG), lambda i, eid: (i, 0, 0)),
            pl.BlockSpec((1, R, R), lambda i, eid: (eid[i], 0, 0)),
            pl.BlockSpec((1, INTER, R), lambda i, eid: (eid[i], 0, 0)),
            pl.BlockSpec((1, R, R), lambda i, eid: (eid[i], 0, 0)),
            pl.BlockSpec((1, INTER, R), lambda i, eid: (eid[i], 0, 0)),
            pl.BlockSpec((R, INTER), lambda i, eid: (0, 0)),
            pl.BlockSpec((1, R, R), lambda i, eid: (eid[i], 0, 0)),
            pl.BlockSpec((1, H, R), lambda i, eid: (eid[i], 0, 0)),
        ],
        out_specs=pl.BlockSpec((TG, H), lambda i, eid: (i, 0)),
    )
    return pl.pallas_call(
        _k2_body,
        grid_spec=grid_spec,
        out_shape=jax.ShapeDtypeStruct((NPAD, H), jnp.bfloat16),
    )(tile_eid, rgru_s, roww3, _b(p['gU']), _b(p['gC']), _b(p['uU']),
      _b(p['uC']), _b(p['Rd']), _b(p['dU']), _b(p['dC']))


def _c_body(outs_ref, invlo_ref, invhi_ref, lo_ref, hi_ref,
            il_v, ih_v, lo_v, hi_v, sem0, sem1):
    cid = lax.axis_index("c")
    sid = lax.axis_index("s")
    wid = sid * 2 + cid
    tbase = wid * TPW

    pltpu.sync_copy(invlo_ref.at[pl.ds(tbase, TPW)], il_v)
    pltpu.sync_copy(invhi_ref.at[pl.ds(tbase, TPW)], ih_v)
    for k in range(TPW // 16):
        sl = pl.ds(k * 16, 16)
        il_v[sl] = jnp.clip(il_v[sl] - 1, 0, NPAD - 1)
        ih_v[sl] = jnp.clip(ih_v[sl] - 1, 0, NPAD - 1)

    for bi in range(TPW // CC):
        c0 = pltpu.async_copy(outs_ref.at[il_v.at[pl.ds(bi * CC, CC)]],
                              lo_v, sem0)
        c1 = pltpu.async_copy(outs_ref.at[ih_v.at[pl.ds(bi * CC, CC)]],
                              hi_v, sem1)
        c0.wait()
        c1.wait()
        pltpu.sync_copy(lo_v, lo_ref.at[pl.ds(tbase + bi * CC, CC)])
        pltpu.sync_copy(hi_v, hi_ref.at[pl.ds(tbase + bi * CC, CC)])


def _c(out32, invlo, invhi):
    mesh = plsc.VectorSubcoreMesh(core_axis_name="c", subcore_axis_name="s")
    f = pl.kernel(
        _c_body,
        out_type=[
            jax.ShapeDtypeStruct((S, H // 2), jnp.int32),
            jax.ShapeDtypeStruct((S, H // 2), jnp.int32),
        ],
        mesh=mesh,
        scratch_types=[
            pltpu.VMEM((TPW,), jnp.int32),
            pltpu.VMEM((TPW,), jnp.int32),
            pltpu.VMEM((CC, H // 2), jnp.int32),
            pltpu.VMEM((CC, H // 2), jnp.int32),
            pltpu.SemaphoreType.DMA,
            pltpu.SemaphoreType.DMA,
        ],
    )
    return f(out32, invlo, invhi)


def _k3_body(ysh_ref, lo_ref, hi_ref, y_ref):
    y_ref[...] = (ysh_ref[...] + lo_ref[...].astype(jnp.float32)
                  + hi_ref[...].astype(jnp.float32))


def _k3(ysh, lo, hi):
    return pl.pallas_call(
        _k3_body,
        grid=(S // T1,),
        in_specs=[
            pl.BlockSpec((T1, H), lambda i: (i, 0)),
            pl.BlockSpec((T1, H), lambda i: (i, 0)),
            pl.BlockSpec((T1, H), lambda i: (i, 0)),
        ],
        out_specs=pl.BlockSpec((T1, H), lambda i: (i, 0)),
        out_shape=jax.ShapeDtypeStruct((S, H), jnp.float32),
    )(ysh, lo, hi)


def kernel(hidden_states, params):
    x = hidden_states.reshape(-1, H)
    wet, rgru, ysh = _k1(x, params)
    gidx, roww, teid, invlo, invhi = _d1(wet)
    rgru_s = _d2(rgru, gidx)
    out_s = _k2(rgru_s, roww.reshape(NT, 1, TG), teid, params)
    out32 = jax.lax.bitcast_convert_type(
        out_s.reshape(NPAD, H // 2, 2), jnp.int32)
    lo32, hi32 = _c(out32, invlo, invhi)
    lo = jax.lax.bitcast_convert_type(lo32, jnp.bfloat16).reshape(S, H)
    hi = jax.lax.bitcast_convert_type(hi32, jnp.bfloat16).reshape(S, H)
    y = _k3(ysh, lo, hi)
    return y.reshape(hidden_states.shape)


# speedup vs baseline: 2.1192x; 2.1192x over previous
import functools

import jax
import jax.numpy as jnp
from jax import lax
from jax.experimental import pallas as pl
from jax.experimental.pallas import tpu as pltpu
from jax.experimental.pallas import tpu_sc as plsc

H = 2048
INTER = 1408
E = 8
R = 256
SH = 2816
S = 2048
T1 = 128
TG = 128
NT = 40
NTP = 48
NPAD = NT * TG
NW = 32
RPW = NPAD // NW
TPW = S // NW
CC = 16


def _mm(a, b):
    return jax.lax.dot_general(a, b, (((1,), (1,)), ((), ())),
                               preferred_element_type=jnp.float32)


def _mmb(a, b):
    return jax.lax.dot_general(a.astype(jnp.bfloat16), b,
                               (((1,), (1,)), ((), ())),
                               preferred_element_type=jnp.float32)


def _silu(x):
    return x * jax.nn.sigmoid(x)


def _b(w):
    return w.astype(jnp.bfloat16)


def _k1_body(x_ref, gate_w_ref, rg_w_ref, ru_w_ref,
             srg_ref, sgu_ref, sgc_ref, sru_ref, suu_ref, suc_ref,
             srd_ref, sdu_ref, sdc_ref,
             wet_ref, rgru_ref, ysh_ref):
    i = pl.program_id(0)
    x = x_ref[...]
    logits = _mm(x, gate_w_ref[...])
    m = jnp.max(logits, axis=-1, keepdims=True)
    ex = jnp.exp(logits - m)
    sc = ex / jnp.sum(ex, axis=-1, keepdims=True)
    eidx = jax.lax.broadcasted_iota(jnp.int32, sc.shape, 1)
    m1 = jnp.max(sc, axis=-1, keepdims=True)
    i1 = jnp.min(jnp.where(sc >= m1, eidx, E), axis=-1, keepdims=True)
    sc2 = jnp.where(eidx == i1, -jnp.inf, sc)
    m2 = jnp.max(sc2, axis=-1, keepdims=True)
    i2 = jnp.min(jnp.where(sc2 >= m2, eidx, E), axis=-1, keepdims=True)
    den = m1 + m2 + 1e-20
    w1 = m1 / den
    w2 = m2 / den
    we = jnp.where(eidx == i1, w1, jnp.where(eidx == i2, w2, 0.0))
    wet_ref[:, pl.ds(i * T1, T1)] = we.T
    xb = x.astype(jnp.bfloat16)
    rgru_ref[:, :R] = _mmb(xb, rg_w_ref[...])
    rgru_ref[:, R:] = _mmb(xb, ru_w_ref[...])
    sg = _silu(_mmb(_mmb(_mmb(xb, srg_ref[...]), sgu_ref[...]), sgc_ref[...]))
    su = _mmb(_mmb(_mmb(xb, sru_ref[...]), suu_ref[...]), suc_ref[...])
    si = sg * su
    ysh_ref[...] = _mmb(_mmb(_mmb(si, srd_ref[...]), sdu_ref[...]), sdc_ref[...])


def _k1(x, p):
    full = lambda shape: pl.BlockSpec(shape, lambda i: (0,) * len(shape))
    grid = S // T1
    return pl.pallas_call(
        _k1_body,
        grid=(grid,),
        in_specs=[
            pl.BlockSpec((T1, H), lambda i: (i, 0)),
            full((E, H)), full((R, H)), full((R, H)),
            full((R, H)), full((R, R)), full((SH, R)),
            full((R, H)), full((R, R)), full((SH, R)),
            full((R, SH)), full((R, R)), full((H, R)),
        ],
        out_specs=[
            pl.BlockSpec((E, S), lambda i: (0, 0)),
            pl.BlockSpec((T1, 2 * R), lambda i: (i, 0)),
            pl.BlockSpec((T1, H), lambda i: (i, 0)),
        ],
        out_shape=[
            jax.ShapeDtypeStruct((E, S), jnp.float32),
            jax.ShapeDtypeStruct((S, 2 * R), jnp.float32),
            jax.ShapeDtypeStruct((S, H), jnp.float32),
        ],
    )(x, p['gate_w'], _b(p['Rg']), _b(p['Ru']),
      _b(p['s_Rg']), _b(p['s_gU']), _b(p['s_gC']), _b(p['s_Ru']),
      _b(p['s_uU']), _b(p['s_uC']), _b(p['s_Rd']), _b(p['s_dU']),
      _b(p['s_dC']))


def _d1_body(wet_ref, gidx_ref, roww_ref, teid_ref, invlo_ref, invhi_ref,
             wet_v, tok_v, w_v, invlo_v, invhi_v, te_v):
    cid = lax.axis_index("c")
    sid = lax.axis_index("s")

    @pl.when(jnp.logical_and(cid == 0, sid <= 9))
    def _work():
        pltpu.sync_copy(wet_ref, wet_v)

        def _count(e):
            def step(j, c):
                v = wet_v[e, pl.ds(j * 16, 16)]
                return c + jnp.sum((v > 0.0).astype(jnp.int32))
            return lax.fori_loop(0, S // 16, step, jnp.int32(0))

        cnts = [_count(e) for e in range(E)]
        ntiles = [(c + (TG - 1)) // TG for c in cnts]
        toff = [jnp.int32(0)]
        for e in range(E):
            toff.append(toff[e] + ntiles[e])
        base = [t * TG for t in toff]

        for e in range(E):
            @pl.when(sid == e)
            def _compact(e=e):
                def zstep(j, _):
                    tok_v[pl.ds(j * 16, 16)] = jnp.zeros((16,), jnp.int32)
                    w_v[pl.ds(j * 16, 16)] = jnp.zeros((16,), jnp.float32)
                    return 0
                lax.fori_loop(0, (S + 16) // 16, zstep, 0)

                def step(j, cnt):
                    v = wet_v[e, pl.ds(j * 16, 16)]
                    mk = v > 0.0
                    toks = j * 16 + lax.iota(jnp.int32, 16)
                    plsc.store_compressed(tok_v.at[pl.ds(cnt, 16)], toks,
                                          mask=mk)
                    plsc.store_compressed(w_v.at[pl.ds(cnt, 16)], v, mask=mk)
                    return cnt + jnp.sum(mk.astype(jnp.int32))
                lax.fori_loop(0, S // 16, step, jnp.int32(0))

                def wstep(k, _):
                    pltpu.sync_copy(
                        tok_v.at[pl.ds(k * TG, TG)],
                        gidx_ref.at[pl.ds(base[e] + k * TG, TG)])
                    pltpu.sync_copy(
                        w_v.at[pl.ds(k * TG, TG)],
                        roww_ref.at[pl.ds(base[e] + k * TG, TG)])
                    return 0
                lax.fori_loop(0, ntiles[e], wstep, 0)

        @pl.when(sid == 8)
        def _inv():
            def step(j, carry):
                acc_s = jnp.zeros((16,), jnp.int32)
                acc_m = jnp.zeros((16,), jnp.int32)
                new = []
                for e in range(E):
                    v = wet_v[e, pl.ds(j * 16, 16)]
                    mk = v > 0.0
                    mi = mk.astype(jnp.int32)
                    pos1 = (base[e] + carry[e]) + plsc.cumsum(mi)
                    p = jnp.where(mk, pos1, 0)
                    acc_s = acc_s + p
                    acc_m = jnp.maximum(acc_m, p)
                    new.append(carry[e] + jnp.sum(mi))
                invlo_v[pl.ds(j * 16, 16)] = acc_s - acc_m
                invhi_v[pl.ds(j * 16, 16)] = acc_m
                return tuple(new)
            lax.fori_loop(0, S // 16, step, (jnp.int32(0),) * E)
            pltpu.sync_copy(invlo_v, invlo_ref)
            pltpu.sync_copy(invhi_v, invhi_ref)

        @pl.when(sid == 9)
        def _teid():
            for j in range(NTP // 16):
                iv = j * 16 + lax.iota(jnp.int32, 16)
                acc = jnp.zeros((16,), jnp.int32)
                for e in range(E):
                    acc = acc + (iv >= toff[e + 1]).astype(jnp.int32)
                te_v[pl.ds(j * 16, 16)] = jnp.minimum(acc, E - 1)
            pltpu.sync_copy(te_v, teid_ref)


def _d1(wet):
    mesh = plsc.VectorSubcoreMesh(core_axis_name="c", subcore_axis_name="s")
    f = pl.kernel(
        _d1_body,
        compiler_params=pltpu.CompilerParams(needs_layout_passes=False),
        out_type=[
            jax.ShapeDtypeStruct((NPAD,), jnp.int32),
            jax.ShapeDtypeStruct((NPAD,), jnp.float32),
            jax.ShapeDtypeStruct((NTP,), jnp.int32),
            jax.ShapeDtypeStruct((S,), jnp.int32),
            jax.ShapeDtypeStruct((S,), jnp.int32),
        ],
        mesh=mesh,
        scratch_types=[
            pltpu.VMEM((E, S), jnp.float32),
            pltpu.VMEM((S + 16,), jnp.int32),
            pltpu.VMEM((S + 16,), jnp.float32),
            pltpu.VMEM((S,), jnp.int32),
            pltpu.VMEM((S,), jnp.int32),
            pltpu.VMEM((NTP,), jnp.int32),
        ],
    )
    return f(wet)


def _d2_body(rgru_ref, gidx_ref, out_ref, idx_v, rows_v, sem0, sem1):
    cid = lax.axis_index("c")
    sid = lax.axis_index("s")
    wid = sid * 2 + cid
    rbase = wid * RPW

    pltpu.sync_copy(gidx_ref.at[pl.ds(rbase, RPW)], idx_v)
    for k in range(RPW // 16):
        idx_v[pl.ds(k * 16, 16)] = jnp.bitwise_and(
            idx_v[pl.ds(k * 16, 16)], S - 1)

    c0 = pltpu.async_copy(rgru_ref.at[idx_v.at[pl.ds(0, 128)]],
                          rows_v.at[pl.ds(0, 128)], sem0)
    c1 = pltpu.async_copy(rgru_ref.at[idx_v.at[pl.ds(128, RPW - 128)]],
                          rows_v.at[pl.ds(128, RPW - 128)], sem1)
    c0.wait()
    c1.wait()
    pltpu.sync_copy(rows_v, out_ref.at[pl.ds(rbase, RPW)])


def _d2(rgru, gidx):
    mesh = plsc.VectorSubcoreMesh(core_axis_name="c", subcore_axis_name="s")
    f = pl.kernel(
        _d2_body,
        out_type=[
            jax.ShapeDtypeStruct((NPAD, 2 * R), jnp.float32),
        ],
        mesh=mesh,
        scratch_types=[
            pltpu.VMEM((RPW,), jnp.int32),
            pltpu.VMEM((RPW, 2 * R), jnp.float32),
            pltpu.SemaphoreType.DMA,
            pltpu.SemaphoreType.DMA,
        ],
    )
    return f(rgru, gidx)[0]


def _k2_body(eid_ref, rgru_ref, roww_ref, gu_ref, gc_ref, uu_ref,
             uc_ref, rd_ref, du_ref, dc_ref, out_ref):
    w = jnp.reshape(roww_ref[0, 0, :], (TG, 1))
    gate = _silu(_mmb(_mmb(rgru_ref[:, :R], gu_ref[0]), gc_ref[0]))
    up = _mmb(_mmb(rgru_ref[:, R:] * w, uu_ref[0]), uc_ref[0])
    inter = gate * up
    out_ref[...] = _mmb(_mmb(_mmb(inter, rd_ref[...]), du_ref[0]), dc_ref[0])


def _k2(rgru_s, roww3, tile_eid, p):
    grid_spec = pltpu.PrefetchScalarGridSpec(
        num_scalar_prefetch=1,
        grid=(NT,),
        in_specs=[
            pl.BlockSpec((TG, 2 * R), lambda i, eid: (i, 0)),
            pl.BlockSpec((1, 1, TG), lambda i, eid: (i, 0, 0)),
            pl.BlockSpec((1, R, R), lambda i, eid: (eid[i], 0, 0)),
            pl.BlockSpec((1, INTER, R), lambda i, eid: (eid[i], 0, 0)),
            pl.BlockSpec((1, R, R), lambda i, eid: (eid[i], 0, 0)),
            pl.BlockSpec((1, INTER, R), lambda i, eid: (eid[i], 0, 0)),
            pl.BlockSpec((R, INTER), lambda i, eid: (0, 0)),
            pl.BlockSpec((1, R, R), lambda i, eid: (eid[i], 0, 0)),
            pl.BlockSpec((1, H, R), lambda i, eid: (eid[i], 0, 0)),
        ],
        out_specs=pl.BlockSpec((TG, H), lambda i, eid: (i, 0)),
    )
    return pl.pallas_call(
        _k2_body,
        grid_spec=grid_spec,
        out_shape=jax.ShapeDtypeStruct((NPAD, H), jnp.float32),
    )(tile_eid, rgru_s, roww3, _b(p['gU']), _b(p['gC']), _b(p['uU']),
      _b(p['uC']), _b(p['Rd']), _b(p['dU']), _b(p['dC']))


def _c_body(outs_ref, invlo_ref, invhi_ref, lo_ref, hi_ref,
            il_v, ih_v, lo_v, hi_v, sem0, sem1):
    cid = lax.axis_index("c")
    sid = lax.axis_index("s")
    wid = sid * 2 + cid
    tbase = wid * TPW

    pltpu.sync_copy(invlo_ref.at[pl.ds(tbase, TPW)], il_v)
    pltpu.sync_copy(invhi_ref.at[pl.ds(tbase, TPW)], ih_v)
    for k in range(TPW // 16):
        sl = pl.ds(k * 16, 16)
        il_v[sl] = jnp.clip(il_v[sl] - 1, 0, NPAD - 1)
        ih_v[sl] = jnp.clip(ih_v[sl] - 1, 0, NPAD - 1)

    for bi in range(TPW // CC):
        c0 = pltpu.async_copy(outs_ref.at[il_v.at[pl.ds(bi * CC, CC)]],
                              lo_v, sem0)
        c1 = pltpu.async_copy(outs_ref.at[ih_v.at[pl.ds(bi * CC, CC)]],
                              hi_v, sem1)
        c0.wait()
        c1.wait()
        pltpu.sync_copy(lo_v, lo_ref.at[pl.ds(tbase + bi * CC, CC)])
        pltpu.sync_copy(hi_v, hi_ref.at[pl.ds(tbase + bi * CC, CC)])


def _c(out_s, invlo, invhi):
    mesh = plsc.VectorSubcoreMesh(core_axis_name="c", subcore_axis_name="s")
    f = pl.kernel(
        _c_body,
        out_type=[
            jax.ShapeDtypeStruct((S, H), jnp.float32),
            jax.ShapeDtypeStruct((S, H), jnp.float32),
        ],
        mesh=mesh,
        scratch_types=[
            pltpu.VMEM((TPW,), jnp.int32),
            pltpu.VMEM((TPW,), jnp.int32),
            pltpu.VMEM((CC, H), jnp.float32),
            pltpu.VMEM((CC, H), jnp.float32),
            pltpu.SemaphoreType.DMA,
            pltpu.SemaphoreType.DMA,
        ],
    )
    return f(out_s, invlo, invhi)


def _k3_body(ysh_ref, lo_ref, hi_ref, y_ref):
    y_ref[...] = ysh_ref[...] + lo_ref[...] + hi_ref[...]


def _k3(ysh, lo, hi):
    return pl.pallas_call(
        _k3_body,
        grid=(S // T1,),
        in_specs=[
            pl.BlockSpec((T1, H), lambda i: (i, 0)),
            pl.BlockSpec((T1, H), lambda i: (i, 0)),
            pl.BlockSpec((T1, H), lambda i: (i, 0)),
        ],
        out_specs=pl.BlockSpec((T1, H), lambda i: (i, 0)),
        out_shape=jax.ShapeDtypeStruct((S, H), jnp.float32),
    )(ysh, lo, hi)


def kernel(hidden_states, params):
    x = hidden_states.reshape(-1, H)
    wet, rgru, ysh = _k1(x, params)
    gidx, roww, teid, invlo, invhi = _d1(wet)
    rgru_s = _d2(rgru, gidx)
    out_s = _k2(rgru_s, roww.reshape(NT, 1, TG), teid, params)
    lo, hi = _c(out_s, invlo, invhi)
    y = _k3(ysh, lo, hi)
    return y.reshape(hidden_states.shape)


# speedup vs baseline: 2.2096x; 1.0427x over previous
import functools

import jax
import jax.numpy as jnp
from jax import lax
from jax.experimental import pallas as pl
from jax.experimental.pallas import tpu as pltpu
from jax.experimental.pallas import tpu_sc as plsc

H = 2048
INTER = 1408
E = 8
R = 256
SH = 2816
S = 2048
T1 = 128
TG = 128
NT = 40
NTP = 48
NPAD = NT * TG
NW = 32
RPW = NPAD // NW
TPW = S // NW
CC = 16


def _mm(a, b):
    return jax.lax.dot_general(a, b, (((1,), (1,)), ((), ())),
                               preferred_element_type=jnp.float32)


def _mmb(a, b):
    return jax.lax.dot_general(a.astype(jnp.bfloat16), b,
                               (((1,), (1,)), ((), ())),
                               preferred_element_type=jnp.float32)


def _silu(x):
    return x * jax.nn.sigmoid(x)


def _b(w):
    return w.astype(jnp.bfloat16)


def _k1a_body(x_ref, gate_w_ref, rg_w_ref, ru_w_ref, wet_ref, rgru_ref):
    i = pl.program_id(0)
    x = x_ref[...]
    logits = _mm(x, gate_w_ref[...])
    m = jnp.max(logits, axis=-1, keepdims=True)
    ex = jnp.exp(logits - m)
    sc = ex / jnp.sum(ex, axis=-1, keepdims=True)
    eidx = jax.lax.broadcasted_iota(jnp.int32, sc.shape, 1)
    m1 = jnp.max(sc, axis=-1, keepdims=True)
    i1 = jnp.min(jnp.where(sc >= m1, eidx, E), axis=-1, keepdims=True)
    sc2 = jnp.where(eidx == i1, -jnp.inf, sc)
    m2 = jnp.max(sc2, axis=-1, keepdims=True)
    i2 = jnp.min(jnp.where(sc2 >= m2, eidx, E), axis=-1, keepdims=True)
    den = m1 + m2 + 1e-20
    w1 = m1 / den
    w2 = m2 / den
    we = jnp.where(eidx == i1, w1, jnp.where(eidx == i2, w2, 0.0))
    wet_ref[:, pl.ds(i * T1, T1)] = we.T
    xb = x.astype(jnp.bfloat16)
    rgru_ref[:, :R] = _mmb(xb, rg_w_ref[...])
    rgru_ref[:, R:] = _mmb(xb, ru_w_ref[...])


def _k1a(x, p):
    full = lambda shape: pl.BlockSpec(shape, lambda i: (0,) * len(shape))
    return pl.pallas_call(
        _k1a_body,
        grid=(S // T1,),
        in_specs=[
            pl.BlockSpec((T1, H), lambda i: (i, 0)),
            full((E, H)), full((R, H)), full((R, H)),
        ],
        out_specs=[
            pl.BlockSpec((E, S), lambda i: (0, 0)),
            pl.BlockSpec((T1, 2 * R), lambda i: (i, 0)),
        ],
        out_shape=[
            jax.ShapeDtypeStruct((E, S), jnp.float32),
            jax.ShapeDtypeStruct((S, 2 * R), jnp.float32),
        ],
    )(x, p['gate_w'], _b(p['Rg']), _b(p['Ru']))


def _k1b_body(x_ref, srg_ref, sgu_ref, sgc_ref, sru_ref, suu_ref, suc_ref,
              srd_ref, sdu_ref, sdc_ref, ysh_ref):
    xb = x_ref[...].astype(jnp.bfloat16)
    sg = _silu(_mmb(_mmb(_mmb(xb, srg_ref[...]), sgu_ref[...]), sgc_ref[...]))
    su = _mmb(_mmb(_mmb(xb, sru_ref[...]), suu_ref[...]), suc_ref[...])
    si = sg * su
    ysh_ref[...] = _mmb(_mmb(_mmb(si, srd_ref[...]), sdu_ref[...]), sdc_ref[...])


def _k1b(x, p):
    full = lambda shape: pl.BlockSpec(shape, lambda i: (0,) * len(shape))
    return pl.pallas_call(
        _k1b_body,
        grid=(S // T1,),
        in_specs=[
            pl.BlockSpec((T1, H), lambda i: (i, 0)),
            full((R, H)), full((R, R)), full((SH, R)),
            full((R, H)), full((R, R)), full((SH, R)),
            full((R, SH)), full((R, R)), full((H, R)),
        ],
        out_specs=pl.BlockSpec((T1, H), lambda i: (i, 0)),
        out_shape=jax.ShapeDtypeStruct((S, H), jnp.float32),
    )(x, _b(p['s_Rg']), _b(p['s_gU']), _b(p['s_gC']), _b(p['s_Ru']),
      _b(p['s_uU']), _b(p['s_uC']), _b(p['s_Rd']), _b(p['s_dU']),
      _b(p['s_dC']))


def _d1_body(wet_ref, gidx_ref, roww_ref, teid_ref, invlo_ref, invhi_ref,
             wet_v, tok_v, w_v, invlo_v, invhi_v, te_v):
    cid = lax.axis_index("c")
    sid = lax.axis_index("s")

    @pl.when(jnp.logical_and(cid == 0, sid <= 9))
    def _work():
        pltpu.sync_copy(wet_ref, wet_v)

        def _count(e):
            def step(j, c):
                v = wet_v[e, pl.ds(j * 16, 16)]
                return c + jnp.sum((v > 0.0).astype(jnp.int32))
            return lax.fori_loop(0, S // 16, step, jnp.int32(0))

        cnts = [_count(e) for e in range(E)]
        ntiles = [(c + (TG - 1)) // TG for c in cnts]
        toff = [jnp.int32(0)]
        for e in range(E):
            toff.append(toff[e] + ntiles[e])
        base = [t * TG for t in toff]

        for e in range(E):
            @pl.when(sid == e)
            def _compact(e=e):
                def zstep(j, _):
                    tok_v[pl.ds(j * 16, 16)] = jnp.zeros((16,), jnp.int32)
                    w_v[pl.ds(j * 16, 16)] = jnp.zeros((16,), jnp.float32)
                    return 0
                lax.fori_loop(0, (S + 16) // 16, zstep, 0)

                def step(j, cnt):
                    v = wet_v[e, pl.ds(j * 16, 16)]
                    mk = v > 0.0
                    toks = j * 16 + lax.iota(jnp.int32, 16)
                    plsc.store_compressed(tok_v.at[pl.ds(cnt, 16)], toks,
                                          mask=mk)
                    plsc.store_compressed(w_v.at[pl.ds(cnt, 16)], v, mask=mk)
                    return cnt + jnp.sum(mk.astype(jnp.int32))
                lax.fori_loop(0, S // 16, step, jnp.int32(0))

                def wstep(k, _):
                    pltpu.sync_copy(
                        tok_v.at[pl.ds(k * TG, TG)],
                        gidx_ref.at[pl.ds(base[e] + k * TG, TG)])
                    pltpu.sync_copy(
                        w_v.at[pl.ds(k * TG, TG)],
                        roww_ref.at[pl.ds(base[e] + k * TG, TG)])
                    return 0
                lax.fori_loop(0, ntiles[e], wstep, 0)

        @pl.when(sid == 8)
        def _inv():
            def step(j, carry):
                acc_s = jnp.zeros((16,), jnp.int32)
                acc_m = jnp.zeros((16,), jnp.int32)
                new = []
                for e in range(E):
                    v = wet_v[e, pl.ds(j * 16, 16)]
                    mk = v > 0.0
                    mi = mk.astype(jnp.int32)
                    pos1 = (base[e] + carry[e]) + plsc.cumsum(mi)
                    p = jnp.where(mk, pos1, 0)
                    acc_s = acc_s + p
                    acc_m = jnp.maximum(acc_m, p)
                    new.append(carry[e] + jnp.sum(mi))
                invlo_v[pl.ds(j * 16, 16)] = acc_s - acc_m
                invhi_v[pl.ds(j * 16, 16)] = acc_m
                return tuple(new)
            lax.fori_loop(0, S // 16, step, (jnp.int32(0),) * E)
            pltpu.sync_copy(invlo_v, invlo_ref)
            pltpu.sync_copy(invhi_v, invhi_ref)

        @pl.when(sid == 9)
        def _teid():
            for j in range(NTP // 16):
                iv = j * 16 + lax.iota(jnp.int32, 16)
                acc = jnp.zeros((16,), jnp.int32)
                for e in range(E):
                    acc = acc + (iv >= toff[e + 1]).astype(jnp.int32)
                te_v[pl.ds(j * 16, 16)] = jnp.minimum(acc, E - 1)
            pltpu.sync_copy(te_v, teid_ref)


def _d1(wet):
    mesh = plsc.VectorSubcoreMesh(core_axis_name="c", subcore_axis_name="s")
    f = pl.kernel(
        _d1_body,
        compiler_params=pltpu.CompilerParams(needs_layout_passes=False),
        out_type=[
            jax.ShapeDtypeStruct((NPAD,), jnp.int32),
            jax.ShapeDtypeStruct((NPAD,), jnp.float32),
            jax.ShapeDtypeStruct((NTP,), jnp.int32),
            jax.ShapeDtypeStruct((S,), jnp.int32),
            jax.ShapeDtypeStruct((S,), jnp.int32),
        ],
        mesh=mesh,
        scratch_types=[
            pltpu.VMEM((E, S), jnp.float32),
            pltpu.VMEM((S + 16,), jnp.int32),
            pltpu.VMEM((S + 16,), jnp.float32),
            pltpu.VMEM((S,), jnp.int32),
            pltpu.VMEM((S,), jnp.int32),
            pltpu.VMEM((NTP,), jnp.int32),
        ],
    )
    return f(wet)


def _d2_body(rgru_ref, gidx_ref, out_ref, idx_v, rows_v, sem0, sem1):
    cid = lax.axis_index("c")
    sid = lax.axis_index("s")
    wid = sid * 2 + cid
    rbase = wid * RPW

    pltpu.sync_copy(gidx_ref.at[pl.ds(rbase, RPW)], idx_v)
    for k in range(RPW // 16):
        idx_v[pl.ds(k * 16, 16)] = jnp.bitwise_and(
            idx_v[pl.ds(k * 16, 16)], S - 1)

    c0 = pltpu.async_copy(rgru_ref.at[idx_v.at[pl.ds(0, 128)]],
                          rows_v.at[pl.ds(0, 128)], sem0)
    c1 = pltpu.async_copy(rgru_ref.at[idx_v.at[pl.ds(128, RPW - 128)]],
                          rows_v.at[pl.ds(128, RPW - 128)], sem1)
    c0.wait()
    c1.wait()
    pltpu.sync_copy(rows_v, out_ref.at[pl.ds(rbase, RPW)])


def _d2(rgru, gidx):
    mesh = plsc.VectorSubcoreMesh(core_axis_name="c", subcore_axis_name="s")
    f = pl.kernel(
        _d2_body,
        out_type=[
            jax.ShapeDtypeStruct((NPAD, 2 * R), jnp.float32),
        ],
        mesh=mesh,
        scratch_types=[
            pltpu.VMEM((RPW,), jnp.int32),
            pltpu.VMEM((RPW, 2 * R), jnp.float32),
            pltpu.SemaphoreType.DMA,
            pltpu.SemaphoreType.DMA,
        ],
    )
    return f(rgru, gidx)[0]


def _k2_body(eid_ref, rgru_ref, roww_ref, gu_ref, gc_ref, uu_ref,
             uc_ref, rd_ref, du_ref, dc_ref, out_ref):
    w = jnp.reshape(roww_ref[0, 0, :], (TG, 1))
    gate = _silu(_mmb(_mmb(rgru_ref[:, :R], gu_ref[0]), gc_ref[0]))
    up = _mmb(_mmb(rgru_ref[:, R:] * w, uu_ref[0]), uc_ref[0])
    inter = gate * up
    out_ref[...] = _mmb(_mmb(_mmb(inter, rd_ref[...]), du_ref[0]), dc_ref[0])


def _k2(rgru_s, roww3, tile_eid, p):
    grid_spec = pltpu.PrefetchScalarGridSpec(
        num_scalar_prefetch=1,
        grid=(NT,),
        in_specs=[
            pl.BlockSpec((TG, 2 * R), lambda i, eid: (i, 0)),
            pl.BlockSpec((1, 1, TG), lambda i, eid: (i, 0, 0)),
            pl.BlockSpec((1, R, R), lambda i, eid: (eid[i], 0, 0)),
            pl.BlockSpec((1, INTER, R), lambda i, eid: (eid[i], 0, 0)),
            pl.BlockSpec((1, R, R), lambda i, eid: (eid[i], 0, 0)),
            pl.BlockSpec((1, INTER, R), lambda i, eid: (eid[i], 0, 0)),
            pl.BlockSpec((R, INTER), lambda i, eid: (0, 0)),
            pl.BlockSpec((1, R, R), lambda i, eid: (eid[i], 0, 0)),
            pl.BlockSpec((1, H, R), lambda i, eid: (eid[i], 0, 0)),
        ],
        out_specs=pl.BlockSpec((TG, H), lambda i, eid: (i, 0)),
    )
    return pl.pallas_call(
        _k2_body,
        grid_spec=grid_spec,
        out_shape=jax.ShapeDtypeStruct((NPAD, H), jnp.float32),
    )(tile_eid, rgru_s, roww3, _b(p['gU']), _b(p['gC']), _b(p['uU']),
      _b(p['uC']), _b(p['Rd']), _b(p['dU']), _b(p['dC']))


def _c_body(outs_ref, invlo_ref, invhi_ref, lo_ref, hi_ref,
            il_v, ih_v, lo_v, hi_v, sem0, sem1):
    cid = lax.axis_index("c")
    sid = lax.axis_index("s")
    wid = sid * 2 + cid
    tbase = wid * TPW

    pltpu.sync_copy(invlo_ref.at[pl.ds(tbase, TPW)], il_v)
    pltpu.sync_copy(invhi_ref.at[pl.ds(tbase, TPW)], ih_v)
    for k in range(TPW // 16):
        sl = pl.ds(k * 16, 16)
        il_v[sl] = jnp.clip(il_v[sl] - 1, 0, NPAD - 1)
        ih_v[sl] = jnp.clip(ih_v[sl] - 1, 0, NPAD - 1)

    for bi in range(TPW // CC):
        c0 = pltpu.async_copy(outs_ref.at[il_v.at[pl.ds(bi * CC, CC)]],
                              lo_v, sem0)
        c1 = pltpu.async_copy(outs_ref.at[ih_v.at[pl.ds(bi * CC, CC)]],
                              hi_v, sem1)
        c0.wait()
        c1.wait()
        pltpu.sync_copy(lo_v, lo_ref.at[pl.ds(tbase + bi * CC, CC)])
        pltpu.sync_copy(hi_v, hi_ref.at[pl.ds(tbase + bi * CC, CC)])


def _c(out_s, invlo, invhi):
    mesh = plsc.VectorSubcoreMesh(core_axis_name="c", subcore_axis_name="s")
    f = pl.kernel(
        _c_body,
        out_type=[
            jax.ShapeDtypeStruct((S, H), jnp.float32),
            jax.ShapeDtypeStruct((S, H), jnp.float32),
        ],
        mesh=mesh,
        scratch_types=[
            pltpu.VMEM((TPW,), jnp.int32),
            pltpu.VMEM((TPW,), jnp.int32),
            pltpu.VMEM((CC, H), jnp.float32),
            pltpu.VMEM((CC, H), jnp.float32),
            pltpu.SemaphoreType.DMA,
            pltpu.SemaphoreType.DMA,
        ],
    )
    return f(out_s, invlo, invhi)


def _k3_body(ysh_ref, lo_ref, hi_ref, y_ref):
    y_ref[...] = ysh_ref[...] + lo_ref[...] + hi_ref[...]


def _k3(ysh, lo, hi):
    return pl.pallas_call(
        _k3_body,
        grid=(S // T1,),
        in_specs=[
            pl.BlockSpec((T1, H), lambda i: (i, 0)),
            pl.BlockSpec((T1, H), lambda i: (i, 0)),
            pl.BlockSpec((T1, H), lambda i: (i, 0)),
        ],
        out_specs=pl.BlockSpec((T1, H), lambda i: (i, 0)),
        out_shape=jax.ShapeDtypeStruct((S, H), jnp.float32),
    )(ysh, lo, hi)


def kernel(hidden_states, params):
    x = hidden_states.reshape(-1, H)
    wet, rgru = _k1a(x, params)
    ysh = _k1b(x, params)
    gidx, roww, teid, invlo, invhi = _d1(wet)
    rgru_s = _d2(rgru, gidx)
    out_s = _k2(rgru_s, roww.reshape(NT, 1, TG), teid, params)
    lo, hi = _c(out_s, invlo, invhi)
    y = _k3(ysh, lo, hi)
    return y.reshape(hidden_states.shape)
